# initial kernel scaffold (unmeasured)
import jax
import jax.numpy as jnp
from jax import lax
from jax.experimental import pallas as pl
from jax.experimental.pallas import tpu as pltpu


def kernel(
    x,
):
    def body(*refs):
        pass

    out_shape = jax.ShapeDtypeStruct(..., jnp.float32)
    return pl.pallas_call(body, out_shape=out_shape)(...)


# baseline (device time: 712574 ns/iter reference)
import os

import jax
import jax.numpy as jnp
from jax import lax
from jax.experimental import pallas as pl
from jax.experimental.pallas import tpu as pltpu

N_DEV = 8

_INTERPRET = (
    pltpu.InterpretParams(detect_races=True, dma_execution_mode="on_wait")
    if os.environ.get("KERNEL_INTERPRET") == "1"
    else False
)


def kernel(x):
    m_per, n = x.shape
    ch = m_per // N_DEV

    def body(
        x_ref,
        out_ref,
        recv_buf,
        x_stage,
        rs_send_sems,
        rs_recv_sems,
        ag_send_sems,
        ag_recv_sems,
        stage_sems,
        credit_rs,
        credit_ag,
    ):
        d = lax.axis_index("i")
        left = (d - 1) % N_DEV
        right = (d + 1) % N_DEV

        barrier = pltpu.get_barrier_semaphore()
        for nbr in (left, right):
            pl.semaphore_signal(
                barrier, inc=1, device_id=(nbr,), device_id_type=pl.DeviceIdType.MESH
            )
        pl.semaphore_wait(barrier, 2)

        def stage(s):
            c = (d - s) % N_DEV
            cp = pltpu.make_async_copy(
                x_ref.at[pl.ds(c * ch, ch), :],
                x_stage.at[s % 2],
                stage_sems.at[s % 2],
            )
            cp.start()
            return cp

        stage(0).wait()
        rdma = pltpu.make_async_remote_copy(
            src_ref=x_stage.at[0],
            dst_ref=recv_buf.at[0],
            send_sem=rs_send_sems.at[0],
            recv_sem=rs_recv_sems.at[0],
            device_id=(right,),
            device_id_type=pl.DeviceIdType.MESH,
        )
        rdma.start()
        rdma.wait()

        for s in range(1, N_DEV - 1):
            stage(s).wait()
            if s >= 2:
                pl.semaphore_wait(credit_rs, 1)
            recv_buf[(s - 1) % 2] = recv_buf[(s - 1) % 2] + x_stage[s % 2]
            rdma = pltpu.make_async_remote_copy(
                src_ref=recv_buf.at[(s - 1) % 2],
                dst_ref=recv_buf.at[s % 2],
                send_sem=rs_send_sems.at[s % 2],
                recv_sem=rs_recv_sems.at[s % 2],
                device_id=(right,),
                device_id_type=pl.DeviceIdType.MESH,
            )
            rdma.start()
            rdma.wait()
            if s <= 5:
                pl.semaphore_signal(
                    credit_rs,
                    inc=1,
                    device_id=(left,),
                    device_id_type=pl.DeviceIdType.MESH,
                )

        stage(7).wait()
        red = (d + 1) % N_DEV
        out_ref[pl.ds(red * ch, ch), :] = recv_buf[0] + x_stage[1]

        for t in range(N_DEV - 1):
            c_snd = (d + 1 - t) % N_DEV
            if t >= 2:
                pl.semaphore_wait(credit_ag, 1)
            rdma = pltpu.make_async_remote_copy(
                src_ref=out_ref.at[pl.ds(c_snd * ch, ch), :],
                dst_ref=out_ref.at[pl.ds(c_snd * ch, ch), :],
                send_sem=ag_send_sems.at[t % 2],
                recv_sem=ag_recv_sems.at[t % 2],
                device_id=(right,),
                device_id_type=pl.DeviceIdType.MESH,
            )
            rdma.start()
            rdma.wait()
            if t <= 4:
                pl.semaphore_signal(
                    credit_ag,
                    inc=1,
                    device_id=(left,),
                    device_id_type=pl.DeviceIdType.MESH,
                )

    return pl.pallas_call(
        body,
        out_shape=jax.ShapeDtypeStruct((m_per, n), x.dtype),
        in_specs=[pl.BlockSpec(memory_space=pl.ANY)],
        out_specs=pl.BlockSpec(memory_space=pltpu.MemorySpace.VMEM),
        scratch_shapes=[
            pltpu.VMEM((2, ch, n), x.dtype),
            pltpu.VMEM((2, ch, n), x.dtype),
            pltpu.SemaphoreType.DMA((2,)),
            pltpu.SemaphoreType.DMA((2,)),
            pltpu.SemaphoreType.DMA((2,)),
            pltpu.SemaphoreType.DMA((2,)),
            pltpu.SemaphoreType.DMA((2,)),
            pltpu.SemaphoreType.REGULAR,
            pltpu.SemaphoreType.REGULAR,
        ],
        compiler_params=pltpu.CompilerParams(
            collective_id=0, vmem_limit_bytes=100 * 1024 * 1024
        ),
        interpret=_INTERPRET,
    )(x)


# device time: 393826 ns/iter; 1.8094x vs baseline; 1.8094x over previous
import os

import jax
import jax.numpy as jnp
from jax import lax
from jax.experimental import pallas as pl
from jax.experimental.pallas import tpu as pltpu

N_DEV = 8

_INTERPRET = (
    pltpu.InterpretParams(detect_races=True, dma_execution_mode="on_wait")
    if os.environ.get("KERNEL_INTERPRET") == "1"
    else False
)


_DO_A = os.environ.get("KERNEL_B_ONLY") != "1"
_DO_B = os.environ.get("KERNEL_A_ONLY") != "1"


def kernel(x):
    m_per, n = x.shape
    ch = m_per // N_DEV
    ch2 = ch // 2

    def body(
        x_ref,
        out_ref,
        buf_a,
        buf_b,
        stg_a,
        stg_b,
        rs_snd_a,
        rs_rcv_a,
        rs_snd_b,
        rs_rcv_b,
        ag_snd_a,
        ag_rcv_a,
        ag_snd_b,
        ag_rcv_b,
        stg_sem_a,
        stg_sem_b,
        cred_rs_a,
        cred_rs_b,
        cred_ag_a,
        cred_ag_b,
    ):
        d = lax.axis_index("i")
        left = (d - 1) % N_DEV
        right = (d + 1) % N_DEV

        barrier = pltpu.get_barrier_semaphore()
        for nbr in (left, right):
            pl.semaphore_signal(
                barrier, inc=1, device_id=(nbr,), device_id_type=pl.DeviceIdType.MESH
            )
        pl.semaphore_wait(barrier, 2)

        def stage_a(s):
            c = (d - s) % N_DEV
            cp = pltpu.make_async_copy(
                x_ref.at[pl.ds(c * ch, ch2), :],
                stg_a.at[s % 2],
                stg_sem_a.at[s % 2],
            )
            cp.start()
            return cp

        def stage_b(s):
            c = (d + s) % N_DEV
            cp = pltpu.make_async_copy(
                x_ref.at[pl.ds(c * ch + ch2, ch2), :],
                stg_b.at[s % 2],
                stg_sem_b.at[s % 2],
            )
            cp.start()
            return cp

        def rs_rdma(s):
            src_a = stg_a.at[0] if s == 0 else buf_a.at[(s - 1) % 2]
            src_b = stg_b.at[0] if s == 0 else buf_b.at[(s - 1) % 2]
            ra = pltpu.make_async_remote_copy(
                src_ref=src_a,
                dst_ref=buf_a.at[s % 2],
                send_sem=rs_snd_a.at[s % 2],
                recv_sem=rs_rcv_a.at[s % 2],
                device_id=(right,),
                device_id_type=pl.DeviceIdType.MESH,
            )
            rb = pltpu.make_async_remote_copy(
                src_ref=src_b,
                dst_ref=buf_b.at[s % 2],
                send_sem=rs_snd_b.at[s % 2],
                recv_sem=rs_rcv_b.at[s % 2],
                device_id=(left,),
                device_id_type=pl.DeviceIdType.MESH,
            )
            if _DO_A:
                ra.start()
            if _DO_B:
                rb.start()
            return ra, rb

        prefetch = os.environ.get("KERNEL_NO_PREFETCH") != "1"
        if prefetch:
            pend_a = [stage_a(0), stage_a(1)] if _DO_A else None
            pend_b = [stage_b(0), stage_b(1)] if _DO_B else None
        else:
            pend_a = [stage_a(0)] if _DO_A else None
            pend_b = [stage_b(0)] if _DO_B else None
        if _DO_A:
            pend_a[0].wait()
        if _DO_B:
            pend_b[0].wait()
        ra, rb = rs_rdma(0)
        if _DO_A:
            ra.wait()
        if _DO_B:
            rb.wait()
        if prefetch:
            if _DO_A:
                pend_a.append(stage_a(2))
            if _DO_B:
                pend_b.append(stage_b(2))

        for s in range(1, N_DEV - 1):
            if not prefetch:
                if _DO_A:
                    pend_a.append(stage_a(s))
                if _DO_B:
                    pend_b.append(stage_b(s))
            if _DO_A:
                pend_a[s].wait()
            if _DO_B:
                pend_b[s].wait()
            if s >= 2:
                if _DO_A:
                    pl.semaphore_wait(cred_rs_a, 1)
                if _DO_B:
                    pl.semaphore_wait(cred_rs_b, 1)
            if _DO_A:
                buf_a[(s - 1) % 2] = buf_a[(s - 1) % 2] + stg_a[s % 2]
            if _DO_B:
                buf_b[(s - 1) % 2] = buf_b[(s - 1) % 2] + stg_b[s % 2]
            ra, rb = rs_rdma(s)
            if _DO_A:
                ra.wait()
            if _DO_B:
                rb.wait()
            if prefetch and s + 2 <= N_DEV - 1:
                if _DO_A:
                    pend_a.append(stage_a(s + 2))
                if _DO_B:
                    pend_b.append(stage_b(s + 2))
            if s <= 5:
                if _DO_A:
                    pl.semaphore_signal(
                        cred_rs_a,
                        inc=1,
                        device_id=(left,),
                        device_id_type=pl.DeviceIdType.MESH,
                    )
                if _DO_B:
                    pl.semaphore_signal(
                        cred_rs_b,
                        inc=1,
                        device_id=(right,),
                        device_id_type=pl.DeviceIdType.MESH,
                    )

        if _DO_A:
            sa = pend_a[N_DEV - 1] if prefetch else stage_a(N_DEV - 1)
            sa.wait()
            red_a = (d + 1) % N_DEV
            out_ref[pl.ds(red_a * ch, ch2), :] = buf_a[0] + stg_a[1]
        if _DO_B:
            sb = pend_b[N_DEV - 1] if prefetch else stage_b(N_DEV - 1)
            sb.wait()
            red_b = (d - 1) % N_DEV
            out_ref[pl.ds(red_b * ch + ch2, ch2), :] = buf_b[0] + stg_b[1]

        for t in range(N_DEV - 1):
            c_a = (d + 1 - t) % N_DEV
            c_b = (d - 1 + t) % N_DEV
            if t >= 2:
                if _DO_A:
                    pl.semaphore_wait(cred_ag_a, 1)
                if _DO_B:
                    pl.semaphore_wait(cred_ag_b, 1)
            ra = pltpu.make_async_remote_copy(
                src_ref=out_ref.at[pl.ds(c_a * ch, ch2), :],
                dst_ref=out_ref.at[pl.ds(c_a * ch, ch2), :],
                send_sem=ag_snd_a.at[t % 2],
                recv_sem=ag_rcv_a.at[t % 2],
                device_id=(right,),
                device_id_type=pl.DeviceIdType.MESH,
            )
            rb = pltpu.make_async_remote_copy(
                src_ref=out_ref.at[pl.ds(c_b * ch + ch2, ch2), :],
                dst_ref=out_ref.at[pl.ds(c_b * ch + ch2, ch2), :],
                send_sem=ag_snd_b.at[t % 2],
                recv_sem=ag_rcv_b.at[t % 2],
                device_id=(left,),
                device_id_type=pl.DeviceIdType.MESH,
            )
            if _DO_A:
                ra.start()
            if _DO_B:
                rb.start()
            if _DO_A:
                ra.wait()
            if _DO_B:
                rb.wait()
            if t <= 4:
                if _DO_A:
                    pl.semaphore_signal(
                        cred_ag_a,
                        inc=1,
                        device_id=(left,),
                        device_id_type=pl.DeviceIdType.MESH,
                    )
                if _DO_B:
                    pl.semaphore_signal(
                        cred_ag_b,
                        inc=1,
                        device_id=(right,),
                        device_id_type=pl.DeviceIdType.MESH,
                    )

    return pl.pallas_call(
        body,
        out_shape=jax.ShapeDtypeStruct((m_per, n), x.dtype),
        in_specs=[pl.BlockSpec(memory_space=pl.ANY)],
        out_specs=pl.BlockSpec(memory_space=pltpu.MemorySpace.VMEM),
        scratch_shapes=[
            pltpu.VMEM((2, ch2, n), x.dtype),
            pltpu.VMEM((2, ch2, n), x.dtype),
            pltpu.VMEM((2, ch2, n), x.dtype),
            pltpu.VMEM((2, ch2, n), x.dtype),
            pltpu.SemaphoreType.DMA((2,)),
            pltpu.SemaphoreType.DMA((2,)),
            pltpu.SemaphoreType.DMA((2,)),
            pltpu.SemaphoreType.DMA((2,)),
            pltpu.SemaphoreType.DMA((2,)),
            pltpu.SemaphoreType.DMA((2,)),
            pltpu.SemaphoreType.DMA((2,)),
            pltpu.SemaphoreType.DMA((2,)),
            pltpu.SemaphoreType.DMA((2,)),
            pltpu.SemaphoreType.DMA((2,)),
            pltpu.SemaphoreType.REGULAR,
            pltpu.SemaphoreType.REGULAR,
            pltpu.SemaphoreType.REGULAR,
            pltpu.SemaphoreType.REGULAR,
        ],
        compiler_params=pltpu.CompilerParams(
            collective_id=0, vmem_limit_bytes=100 * 1024 * 1024
        ),
        interpret=_INTERPRET,
    )(x)


# device time: 358144 ns/iter; 1.9896x vs baseline; 1.0996x over previous
import os

import jax
import jax.numpy as jnp
from jax import lax
from jax.experimental import pallas as pl
from jax.experimental.pallas import tpu as pltpu

N_DEV = 8

_INTERPRET = (
    pltpu.InterpretParams(detect_races=True, dma_execution_mode="on_wait")
    if os.environ.get("KERNEL_INTERPRET") == "1"
    else False
)

_MESH = pl.DeviceIdType.MESH


def kernel(x):
    m_per, n = x.shape
    ch = m_per // N_DEV
    ch2 = ch // 2
    ch4 = ch // 4

    n_q = 4

    def body(x_ref, out_ref, *scr):
        bufs = scr[0:4]
        stgs = scr[4:8]
        rs_snd = [scr[8 + 5 * q + 0] for q in range(n_q)]
        rs_rcv = [scr[8 + 5 * q + 1] for q in range(n_q)]
        ag_snd = [scr[8 + 5 * q + 2] for q in range(n_q)]
        ag_rcv = [scr[8 + 5 * q + 3] for q in range(n_q)]
        stg_sem = [scr[8 + 5 * q + 4] for q in range(n_q)]
        cred_rs = scr[28:32]
        cred_ag = scr[32:36]

        d = lax.axis_index("i")
        r = jnp.where(d < 4, d, 11 - d)
        nxt = jnp.where(d < 4, jnp.where(d == 3, 7, d + 1), jnp.where(d == 4, 0, d - 1))
        prv = jnp.where(d < 4, jnp.where(d == 0, 4, d - 1), jnp.where(d == 7, 3, d + 1))

        sub = [
            (nxt, prv, -1, 0 * ch4),
            (prv, nxt, +1, ch2),
            (nxt, prv, -1, 1 * ch4),
            (prv, nxt, +1, ch2 + ch4),
        ]

        barrier = pltpu.get_barrier_semaphore()
        for nbr in (nxt, prv):
            pl.semaphore_signal(barrier, inc=1, device_id=(nbr,), device_id_type=_MESH)
        pl.semaphore_wait(barrier, 2)

        def stage(q, s):
            _, _, sign, off = sub[q]
            c = (r + sign * s) % N_DEV
            cp = pltpu.make_async_copy(
                x_ref.at[pl.ds(c * ch + off, ch4), :],
                stgs[q].at[s % 2],
                stg_sem[q].at[s % 2],
            )
            cp.start()
            return cp

        def rs_start(q, s):
            tgt, _, _, _ = sub[q]
            src = stgs[q].at[0] if s == 0 else bufs[q].at[(s - 1) % 2]
            rd = pltpu.make_async_remote_copy(
                src_ref=src,
                dst_ref=bufs[q].at[s % 2],
                send_sem=rs_snd[q].at[s % 2],
                recv_sem=rs_rcv[q].at[s % 2],
                device_id=(tgt,),
                device_id_type=_MESH,
            )
            rd.start()
            return rd

        pend_stg = [[stage(q, 0), stage(q, 1)] for q in range(n_q)]
        rs_desc = [[] for _ in range(n_q)]
        for q in range(n_q):
            pend_stg[q][0].wait()
            rs_desc[q].append(rs_start(q, 0))

        for s in range(1, N_DEV - 1):
            for q in range(n_q):
                tgt, src_dev, _, _ = sub[q]
                rs_desc[q][s - 1].wait_recv()
                pend_stg[q][s].wait()
                bufs[q][(s - 1) % 2] = bufs[q][(s - 1) % 2] + stgs[q][s % 2]
                rs_desc[q][s - 1].wait_send()
                pend_stg[q].append(stage(q, s + 1))
                if s >= 2:
                    pl.semaphore_signal(
                        cred_rs[q], inc=1, device_id=(src_dev,), device_id_type=_MESH
                    )
                    pl.semaphore_wait(cred_rs[q], 1)
                rs_desc[q].append(rs_start(q, s))

        for q in range(n_q):
            _, _, sign, off = sub[q]
            rs_desc[q][N_DEV - 2].wait_recv()
            pend_stg[q][N_DEV - 1].wait()
            red = (r - sign) % N_DEV
            out_ref[pl.ds(red * ch + off, ch4), :] = bufs[q][0] + stgs[q][1]
            rs_desc[q][N_DEV - 2].wait_send()

        def ag_start(q, t):
            tgt, _, sign, off = sub[q]
            c = (r + sign * (t - 1)) % N_DEV
            rd = pltpu.make_async_remote_copy(
                src_ref=out_ref.at[pl.ds(c * ch + off, ch4), :],
                dst_ref=out_ref.at[pl.ds(c * ch + off, ch4), :],
                send_sem=ag_snd[q].at[t % 2],
                recv_sem=ag_rcv[q].at[t % 2],
                device_id=(tgt,),
                device_id_type=_MESH,
            )
            rd.start()
            return rd

        ag_desc = [[] for _ in range(n_q)]
        for q in range(n_q):
            ag_desc[q].append(ag_start(q, 0))

        for t in range(1, N_DEV - 1):
            for q in range(n_q):
                _, src_dev, _, _ = sub[q]
                ag_desc[q][t - 1].wait_recv()
                ag_desc[q][t - 1].wait_send()
                if 1 <= t <= 5:
                    pl.semaphore_signal(
                        cred_ag[q], inc=1, device_id=(src_dev,), device_id_type=_MESH
                    )
                if t >= 2:
                    pl.semaphore_wait(cred_ag[q], 1)
                ag_desc[q].append(ag_start(q, t))

        for q in range(n_q):
            ag_desc[q][N_DEV - 2].wait_recv()
            ag_desc[q][N_DEV - 2].wait_send()

    scratch = (
        [pltpu.VMEM((2, ch4, n), x.dtype) for _ in range(4)]
        + [pltpu.VMEM((2, ch4, n), x.dtype) for _ in range(4)]
        + [pltpu.SemaphoreType.DMA((2,)) for _ in range(20)]
        + [pltpu.SemaphoreType.REGULAR for _ in range(8)]
    )
    return pl.pallas_call(
        body,
        out_shape=jax.ShapeDtypeStruct((m_per, n), x.dtype),
        in_specs=[pl.BlockSpec(memory_space=pl.ANY)],
        out_specs=pl.BlockSpec(memory_space=pltpu.MemorySpace.VMEM),
        scratch_shapes=scratch,
        compiler_params=pltpu.CompilerParams(
            collective_id=0, vmem_limit_bytes=100 * 1024 * 1024
        ),
        interpret=_INTERPRET,
    )(x)
